# baseline (speedup 1.0000x reference)
import dataclasses
import functools

import jax
import jax.numpy as jnp
from jax import lax
from jax.experimental import pallas as pl
from jax.experimental.pallas import tpu as pltpu
from jax.experimental.pallas import tpu_sc as plsc

B = 128
S = 64
H = 1024
DEPTH = 64
NW = 32
BPW = B // NW
L = 16
KB = 32
DMARK = B * S
ZMARK = B * S + 1


def _proj_tc_kernel(x_ref, w_ref, d_ref, proj_ref, dproj_ref):
    proj_ref[...] = jnp.dot(x_ref[...], w_ref[...],
                            preferred_element_type=jnp.float32)
    dproj_ref[...] = jnp.dot(d_ref[...], w_ref[...],
                             preferred_element_type=jnp.float32)


def _seq_tc_kernel(*refs):
    hs_refs = refs[0:KB]
    pr_refs = refs[KB:2 * KB]
    (wa_ref, ba_ref, dp_ref, dv_ref, so_ref, stack_ref, x_ref,
     pend_ref) = refs[2 * KB:]
    stack_ref[...] = jnp.zeros_like(stack_ref)
    iot_c = lax.broadcasted_iota(jnp.int32, (DEPTH, 1), 0)
    iot_r = lax.broadcasted_iota(jnp.int32, (1, DEPTH), 1)
    dw = dp_ref[0:1, 0:1]
    dvrow = dv_ref[0:1, :]
    pid = pl.program_id(0)

    def step(t, carry):
        ls, srcs, ptrs, pslots = carry
        ws = []
        for k in range(KB):
            el = jnp.exp(ls[k])
            Z = jnp.sum(el, axis=0, keepdims=True)
            ws.append(el / Z)
        for k in range(KB):
            sk = stack_ref[pl.ds(k * DEPTH, DEPTH), :]
            sk = jnp.where(iot_c == pslots[k], pend_ref[k:k + 1, :], sk)
            stack_ref[pl.ds(k * DEPTH, DEPTH), :] = sk
            ctx = jnp.sum(sk * ws[k], axis=0, keepdims=True)
            x_ref[pl.ds(k, 1), :] = hs_refs[k][t] + ctx
        alog = jnp.dot(x_ref[...], wa_ref[...],
                       preferred_element_type=jnp.float32) + ba_ref[0:1, :]

        nls, nsrcs, nptrs, npslots = [], [], [], []
        for k in range(KB):
            a0 = alog[k:k + 1, 0:1]
            a1 = alog[k:k + 1, 1:2]
            a2 = alog[k:k + 1, 2:3]
            is_push = (a0 >= a1) & (a0 >= a2)
            is_pop = jnp.logical_not(is_push) & (a1 >= a2)
            do_w = is_push | is_pop
            ptr = ptrs[k]
            slot = jnp.where(is_push, (ptr - 1) & (DEPTH - 1),
                             ptr & (DEPTH - 1))
            mc = (iot_c == slot) & do_w
            mr = (iot_r == slot) & do_w
            prb = pr_refs[k][t]
            hwt = prb[:, 0:1]
            rb = (pid * KB + k) * S
            nls.append(jnp.where(mc, jnp.where(is_push, hwt, dw), ls[k]))
            nsrcs.append(jnp.where(mr, jnp.where(is_push, rb + t, DMARK),
                                   srcs[k]))
            ht = hs_refs[k][t]
            pend_ref[k:k + 1, :] = jnp.where(is_push, ht, dvrow)
            npslots.append(jnp.where(do_w, slot, -1))
            nptrs.append(ptr + jnp.where(is_push, -1,
                                         jnp.where(is_pop, 1, 0)))
        return (tuple(nls), tuple(nsrcs), tuple(nptrs), tuple(npslots))

    init = (tuple(jnp.zeros((DEPTH, 1), jnp.float32) for _ in range(KB)),
            tuple(jnp.full((1, DEPTH), ZMARK, jnp.int32) for _ in range(KB)),
            tuple(jnp.zeros((1, 1), jnp.int32) for _ in range(KB)),
            tuple(jnp.full((1, 1), -1, jnp.int32) for _ in range(KB)))
    _, srcs, ptrs, _ = lax.fori_loop(0, S, step, init)
    for k in range(KB):
        so_ref[k, :, 0:DEPTH] = srcs[k]
        so_ref[k, :, DEPTH:2 * DEPTH] = jnp.broadcast_to(ptrs[k], (1, DEPTH))


def _seq_call(hidden_state, proj, wact_pad, bact_pad, dproj, d_pad,
              interpret=False):
    return pl.pallas_call(
        _seq_tc_kernel,
        grid=(B // KB,),
        in_specs=(
            [pl.BlockSpec((None, S, 1, H),
                          functools.partial(lambda k, i: (i * KB + k, 0, 0, 0),
                                            k))
             for k in range(KB)]
            + [pl.BlockSpec((None, S, 1, 128),
                            functools.partial(
                                lambda k, i: (i * KB + k, 0, 0, 0), k))
               for k in range(KB)]
            + [pl.BlockSpec((H, 128), lambda i: (0, 0)),
               pl.BlockSpec((8, 128), lambda i: (0, 0)),
               pl.BlockSpec((8, 128), lambda i: (0, 0)),
               pl.BlockSpec((8, H), lambda i: (0, 0))]
        ),
        out_specs=pl.BlockSpec((KB, 1, 128), lambda i: (i, 0, 0)),
        out_shape=jax.ShapeDtypeStruct((B, 1, 128), jnp.int32),
        scratch_shapes=[pltpu.VMEM((KB * DEPTH, H), jnp.float32),
                        pltpu.VMEM((KB, H), jnp.float32),
                        pltpu.VMEM((KB, H), jnp.float32)],
        interpret=interpret,
    )(*([hidden_state.reshape(B, S, 1, H)] * KB
        + [proj.reshape(B, S, 1, 128)] * KB
        + [wact_pad, bact_pad, dproj, d_pad]))


def _sc_kernel(sout_hbm, hs_hbm, dvec_hbm, out_hbm,
               sv, fsrcv, gidx, dvec, zvec, rows):
    wid = lax.axis_index("s") * 2 + lax.axis_index("c")
    pltpu.sync_copy(dvec_hbm, dvec)

    @pl.loop(0, H, step=L)
    def _(c):
        zvec[pl.ds(c, L)] = jnp.zeros((L,), jnp.float32)

    iota = lax.iota(jnp.int32, L)

    @pl.loop(0, BPW)
    def _batch(bi):
        b = wid * BPW + bi
        pltpu.sync_copy(sout_hbm.at[b], sv)
        pv = sv[pl.ds(DEPTH, L)]
        ptr = pv[0]

        for j in range(4):
            idxs = (ptr + iota + (L * j)) & (DEPTH - 1)
            fs = plsc.load_gather(sv, [idxs])
            fsrcv[pl.ds(L * j, L)] = fs
            gidx[pl.ds(L * j, L)] = jnp.minimum(fs, B * S - 1)

        pltpu.sync_copy(hs_hbm.at[gidx], rows)

        for j in range(4):
            fv = fsrcv[pl.ds(L * j, L)]
            for k in range(L):
                i = L * j + k

                @pl.when(fv[k] == DMARK)
                def _():
                    @pl.loop(0, H, step=L)
                    def _(c):
                        rows[i, pl.ds(c, L)] = dvec[pl.ds(c, L)]

                @pl.when(fv[k] == ZMARK)
                def _():
                    @pl.loop(0, H, step=L)
                    def _(c):
                        rows[i, pl.ds(c, L)] = zvec[pl.ds(c, L)]

        pltpu.sync_copy(rows, out_hbm.at[pl.ds(b * DEPTH, DEPTH)])


def kernel(hidden_state, W_attn, b_attn, W_act, b_act, D):
    del b_attn
    hs_flat = hidden_state.reshape(B * S, H)
    w_cat = jnp.concatenate([W_attn, W_act], axis=1)
    w_pad = jnp.pad(w_cat, ((0, 0), (0, 124)))
    wact_pad = jnp.pad(W_act, ((0, 0), (0, 125)))
    bact_pad = jnp.pad(b_act, (0, 125)).reshape(1, 128)
    bact_pad = jnp.pad(bact_pad, ((0, 7), (0, 0)))
    d_pad = jnp.pad(D, ((0, 7), (0, 0)))

    proj, dproj = pl.pallas_call(
        _proj_tc_kernel,
        grid=(16,),
        in_specs=[
            pl.BlockSpec((B * S // 16, H), lambda i: (i, 0)),
            pl.BlockSpec((H, 128), lambda i: (0, 0)),
            pl.BlockSpec((8, H), lambda i: (0, 0)),
        ],
        out_specs=[
            pl.BlockSpec((B * S // 16, 128), lambda i: (i, 0)),
            pl.BlockSpec((8, 128), lambda i: (0, 0)),
        ],
        out_shape=[
            jax.ShapeDtypeStruct((B * S, 128), jnp.float32),
            jax.ShapeDtypeStruct((8, 128), jnp.float32),
        ],
    )(hs_flat, w_pad, d_pad)

    sout = _seq_call(hidden_state, proj, wact_pad, bact_pad, dproj, d_pad)
    sout = sout.reshape(B, 128)

    dvec = D[0]

    cp = pltpu.CompilerParams()
    if "needs_layout_passes" in pltpu.CompilerParams.__dataclass_fields__:
        cp = dataclasses.replace(cp, needs_layout_passes=False)
    sc = functools.partial(
        pl.kernel,
        out_type=jax.ShapeDtypeStruct((B * S, H), jnp.float32),
        compiler_params=cp,
        mesh=plsc.VectorSubcoreMesh(core_axis_name="c", subcore_axis_name="s"),
        scratch_types=[
            pltpu.VMEM((128,), jnp.int32),
            pltpu.VMEM((DEPTH,), jnp.int32),
            pltpu.VMEM((DEPTH,), jnp.int32),
            pltpu.VMEM((H,), jnp.float32),
            pltpu.VMEM((H,), jnp.float32),
            pltpu.VMEM((DEPTH, H), jnp.float32),
        ],
    )(_sc_kernel)

    out = sc(sout, hs_flat, dvec)
    return out.reshape(B, DEPTH, H)


# speedup vs baseline: 1.0336x; 1.0336x over previous
import dataclasses
import functools

import jax
import jax.numpy as jnp
from jax import lax
from jax.experimental import pallas as pl
from jax.experimental.pallas import tpu as pltpu
from jax.experimental.pallas import tpu_sc as plsc

B = 128
S = 64
H = 1024
DEPTH = 64
NW = 32
BPW = B // NW
L = 16
KB = 32
DMARK = B * S
ZMARK = B * S + 1


def _proj_tc_kernel(x_ref, w_ref, d_ref, proj_ref, dproj_ref):
    proj_ref[...] = jnp.dot(x_ref[...], w_ref[...],
                            preferred_element_type=jnp.float32)
    dproj_ref[...] = jnp.dot(d_ref[...], w_ref[...],
                             preferred_element_type=jnp.float32)


def _seq_tc_kernel(*refs):
    hs_refs = refs[0:KB]
    pr_refs = refs[KB:2 * KB]
    wa_ref, ba_ref, dp_ref, dv_ref, so_ref, stack_ref, x_ref = refs[2 * KB:]
    stack_ref[...] = jnp.zeros_like(stack_ref)
    iot_c = lax.broadcasted_iota(jnp.int32, (DEPTH, 1), 0)
    iot_r = lax.broadcasted_iota(jnp.int32, (1, DEPTH), 1)
    dw = dp_ref[0:1, 0:1]
    dvrow = dv_ref[0:1, :]
    pid = pl.program_id(0)

    def step(t, carry):
        ls, srcs, ptrs = carry
        ws = []
        for k in range(KB):
            el = jnp.exp(ls[k])
            Z = jnp.sum(el, axis=0, keepdims=True)
            ws.append(el / Z)
        for k in range(KB):
            sk = stack_ref[pl.ds(k * DEPTH, DEPTH), :]
            ctx = jnp.sum(sk * ws[k], axis=0, keepdims=True)
            x_ref[pl.ds(k, 1), :] = hs_refs[k][t] + ctx
        alog = jnp.dot(x_ref[...], wa_ref[...],
                       preferred_element_type=jnp.float32) + ba_ref[0:1, :]

        nls, nsrcs, nptrs = [], [], []
        for k in range(KB):
            a0 = alog[k:k + 1, 0:1]
            a1 = alog[k:k + 1, 1:2]
            a2 = alog[k:k + 1, 2:3]
            is_push = (a0 >= a1) & (a0 >= a2)
            is_pop = jnp.logical_not(is_push) & (a1 >= a2)
            do_w = is_push | is_pop
            ptr = ptrs[k]
            slot = jnp.where(is_push, (ptr - 1) & (DEPTH - 1),
                             ptr & (DEPTH - 1))
            mc = (iot_c == slot) & do_w
            mr = (iot_r == slot) & do_w
            prb = pr_refs[k][t]
            hwt = prb[:, 0:1]
            rb = (pid * KB + k) * S
            nls.append(jnp.where(mc, jnp.where(is_push, hwt, dw), ls[k]))
            nsrcs.append(jnp.where(mr, jnp.where(is_push, rb + t, DMARK),
                                   srcs[k]))
            ht = hs_refs[k][t]
            newrow = jnp.where(is_push, ht, dvrow)
            sk = stack_ref[pl.ds(k * DEPTH, DEPTH), :]
            stack_ref[pl.ds(k * DEPTH, DEPTH), :] = jnp.where(mc, newrow, sk)
            nptrs.append(ptr + jnp.where(is_push, -1,
                                         jnp.where(is_pop, 1, 0)))
        return (tuple(nls), tuple(nsrcs), tuple(nptrs))

    init = (tuple(jnp.zeros((DEPTH, 1), jnp.float32) for _ in range(KB)),
            tuple(jnp.full((1, DEPTH), ZMARK, jnp.int32) for _ in range(KB)),
            tuple(jnp.zeros((1, 1), jnp.int32) for _ in range(KB)))
    _, srcs, ptrs = lax.fori_loop(0, S, step, init)
    for k in range(KB):
        so_ref[k, :, 0:DEPTH] = srcs[k]
        so_ref[k, :, DEPTH:2 * DEPTH] = jnp.broadcast_to(ptrs[k], (1, DEPTH))


def _seq_call(hidden_state, proj, wact_pad, bact_pad, dproj, d_pad,
              interpret=False):
    return pl.pallas_call(
        _seq_tc_kernel,
        grid=(B // KB,),
        in_specs=(
            [pl.BlockSpec((None, S, 1, H),
                          functools.partial(lambda k, i: (i * KB + k, 0, 0, 0),
                                            k))
             for k in range(KB)]
            + [pl.BlockSpec((None, S, 1, 128),
                            functools.partial(
                                lambda k, i: (i * KB + k, 0, 0, 0), k))
               for k in range(KB)]
            + [pl.BlockSpec((H, 128), lambda i: (0, 0)),
               pl.BlockSpec((8, 128), lambda i: (0, 0)),
               pl.BlockSpec((8, 128), lambda i: (0, 0)),
               pl.BlockSpec((8, H), lambda i: (0, 0))]
        ),
        out_specs=pl.BlockSpec((KB, 1, 128), lambda i: (i, 0, 0)),
        out_shape=jax.ShapeDtypeStruct((B, 1, 128), jnp.int32),
        scratch_shapes=[pltpu.VMEM((KB * DEPTH, H), jnp.float32),
                        pltpu.VMEM((KB, H), jnp.float32)],
        interpret=interpret,
    )(*([hidden_state.reshape(B, S, 1, H)] * KB
        + [proj.reshape(B, S, 1, 128)] * KB
        + [wact_pad, bact_pad, dproj, d_pad]))


def _sc_kernel(sout_hbm, hs_hbm, dvec_hbm, out_hbm,
               sv, fsrcv, gidx, dvec, zvec, rows):
    wid = lax.axis_index("s") * 2 + lax.axis_index("c")
    pltpu.sync_copy(dvec_hbm, dvec)

    @pl.loop(0, H, step=L)
    def _(c):
        zvec[pl.ds(c, L)] = jnp.zeros((L,), jnp.float32)

    iota = lax.iota(jnp.int32, L)

    @pl.loop(0, BPW)
    def _batch(bi):
        b = wid * BPW + bi
        pltpu.sync_copy(sout_hbm.at[b], sv)
        pv = sv[pl.ds(DEPTH, L)]
        ptr = pv[0]

        for j in range(4):
            idxs = (ptr + iota + (L * j)) & (DEPTH - 1)
            fs = plsc.load_gather(sv, [idxs])
            fsrcv[pl.ds(L * j, L)] = fs
            gidx[pl.ds(L * j, L)] = jnp.minimum(fs, B * S - 1)

        pltpu.sync_copy(hs_hbm.at[gidx], rows)

        for j in range(4):
            fv = fsrcv[pl.ds(L * j, L)]
            for k in range(L):
                i = L * j + k

                @pl.when(fv[k] == DMARK)
                def _():
                    @pl.loop(0, H, step=L)
                    def _(c):
                        rows[i, pl.ds(c, L)] = dvec[pl.ds(c, L)]

                @pl.when(fv[k] == ZMARK)
                def _():
                    @pl.loop(0, H, step=L)
                    def _(c):
                        rows[i, pl.ds(c, L)] = zvec[pl.ds(c, L)]

        pltpu.sync_copy(rows, out_hbm.at[pl.ds(b * DEPTH, DEPTH)])


def kernel(hidden_state, W_attn, b_attn, W_act, b_act, D):
    del b_attn
    hs_flat = hidden_state.reshape(B * S, H)
    w_cat = jnp.concatenate([W_attn, W_act], axis=1)
    w_pad = jnp.pad(w_cat, ((0, 0), (0, 124)))
    wact_pad = jnp.pad(W_act, ((0, 0), (0, 125)))
    bact_pad = jnp.pad(b_act, (0, 125)).reshape(1, 128)
    bact_pad = jnp.pad(bact_pad, ((0, 7), (0, 0)))
    d_pad = jnp.pad(D, ((0, 7), (0, 0)))

    proj, dproj = pl.pallas_call(
        _proj_tc_kernel,
        grid=(16,),
        in_specs=[
            pl.BlockSpec((B * S // 16, H), lambda i: (i, 0)),
            pl.BlockSpec((H, 128), lambda i: (0, 0)),
            pl.BlockSpec((8, H), lambda i: (0, 0)),
        ],
        out_specs=[
            pl.BlockSpec((B * S // 16, 128), lambda i: (i, 0)),
            pl.BlockSpec((8, 128), lambda i: (0, 0)),
        ],
        out_shape=[
            jax.ShapeDtypeStruct((B * S, 128), jnp.float32),
            jax.ShapeDtypeStruct((8, 128), jnp.float32),
        ],
    )(hs_flat, w_pad, d_pad)

    sout = _seq_call(hidden_state, proj, wact_pad, bact_pad, dproj, d_pad)
    sout = sout.reshape(B, 128)

    dvec = D[0]

    cp = pltpu.CompilerParams()
    if "needs_layout_passes" in pltpu.CompilerParams.__dataclass_fields__:
        cp = dataclasses.replace(cp, needs_layout_passes=False)
    sc = functools.partial(
        pl.kernel,
        out_type=jax.ShapeDtypeStruct((B * S, H), jnp.float32),
        compiler_params=cp,
        mesh=plsc.VectorSubcoreMesh(core_axis_name="c", subcore_axis_name="s"),
        scratch_types=[
            pltpu.VMEM((128,), jnp.int32),
            pltpu.VMEM((DEPTH,), jnp.int32),
            pltpu.VMEM((DEPTH,), jnp.int32),
            pltpu.VMEM((H,), jnp.float32),
            pltpu.VMEM((H,), jnp.float32),
            pltpu.VMEM((DEPTH, H), jnp.float32),
        ],
    )(_sc_kernel)

    out = sc(sout, hs_flat, dvec)
    return out.reshape(B, DEPTH, H)
